# baseline (speedup 1.0000x reference)
import jax
import jax.numpy as jnp
from jax import lax
from jax.experimental import pallas as pl
from jax.experimental.pallas import tpu as pltpu
from jax.experimental.pallas import tpu_sc as plsc

_B, _P, _D, _L = 64, 1024, 256, 80
_N = _B * _P
_NC, _NS = 2, 16
_NW = _NC * _NS
_RPW = _N // _NW
_BLK = 128
_NBLK = _RPW // _BLK
_LANES = 16


def _sc_encoder(xy_hbm, lab_hbm, tab_hbm, w_hbm, b_hbm, out_hbm,
                tab_v, xy_s, x_v, y_v, lab_v, w_v, b_v, out_v0, out_v1,
                sem0, sem1):
    wid = lax.axis_index("s") * _NC + lax.axis_index("c")
    base = wid * _RPW

    pltpu.sync_copy(xy_hbm.at[pl.ds(2 * base, 2 * _RPW)], xy_s)
    pltpu.sync_copy(lab_hbm.at[pl.ds(base, _RPW)], lab_v.at[pl.ds(0, _RPW)])
    pltpu.sync_copy(tab_hbm, tab_v)
    pltpu.sync_copy(w_hbm, w_v)
    pltpu.sync_copy(b_hbm, b_v)

    iota = lax.iota(jnp.int32, _LANES)
    idx_even = iota * 2
    idx_odd = idx_even + 1

    @plsc.parallel_loop(0, _RPW // _LANES, unroll=2)
    def deint(g):
        bg = jnp.broadcast_to(g * (2 * _LANES), (_LANES,))
        xe = plsc.load_gather(xy_s, [bg + idx_even])
        ye = plsc.load_gather(xy_s, [bg + idx_odd])
        x_v[pl.ds(g * _LANES, _LANES)] = xe
        y_v[pl.ds(g * _LANES, _LANES)] = ye

    br = [b_v[pl.ds(jb * _LANES, _LANES)] for jb in range(_D // _LANES)]

    @plsc.parallel_loop(0, _L, unroll=2)
    def fold(r):
        rb = r * _D
        for jb in range(_D // _LANES):
            sl = pl.ds(rb + jb * _LANES, _LANES)
            tab_v[sl] = tab_v[sl] + br[jb]

    w0r = [w_v[pl.ds(jb * _LANES, _LANES)] for jb in range(_D // _LANES)]
    w1r = [w_v[pl.ds(_D + jb * _LANES, _LANES)] for jb in range(_D // _LANES)]

    def compute_block(buf, off):
        @plsc.parallel_loop(0, _BLK, unroll=2)
        def body(i):
            p = i + off
            xv = x_v[pl.ds(p, _LANES)]
            yv = y_v[pl.ds(p, _LANES)]
            lv = lab_v[pl.ds(p, _LANES)]
            xs = xv[0]
            ys = yv[0]
            rowbase = lv[0] * _D
            for jb in range(_D // _LANES):
                tbl = tab_v[pl.ds(rowbase + jb * _LANES, _LANES)]
                val = tbl + xs * w0r[jb] + ys * w1r[jb]
                buf[i, pl.ds(jb * _LANES, _LANES)] = val

    def issue(buf, off, sem):
        pltpu.async_copy(buf, out_hbm.at[pl.ds(base + off, _BLK)], sem)

    def drain(buf, off, sem):
        pltpu.make_async_copy(buf, out_hbm.at[pl.ds(base + off, _BLK)], sem).wait()

    compute_block(out_v0, 0)
    issue(out_v0, 0, sem0)
    compute_block(out_v1, _BLK)
    issue(out_v1, _BLK, sem1)

    def sb_body(sb, carry):
        off = sb * (2 * _BLK)
        drain(out_v0, off, sem0)
        compute_block(out_v0, off)
        issue(out_v0, off, sem0)
        drain(out_v1, off + _BLK, sem1)
        compute_block(out_v1, off + _BLK)
        issue(out_v1, off + _BLK, sem1)
        return carry

    lax.fori_loop(1, _NBLK // 2, sb_body, 0)
    drain(out_v0, 0, sem0)
    drain(out_v1, _BLK, sem1)


def kernel(points, labels, W_pos, b_pos, label_table):
    xy = points.reshape(-1).astype(jnp.float32)
    lab = labels.reshape(-1).astype(jnp.int32)
    tab = label_table.reshape(-1).astype(jnp.float32)
    w = W_pos.reshape(-1).astype(jnp.float32)
    b = b_pos.astype(jnp.float32)

    mesh = plsc.VectorSubcoreMesh(core_axis_name="c", subcore_axis_name="s")
    enc = pl.kernel(
        _sc_encoder,
        out_type=jax.ShapeDtypeStruct((_N, _D), jnp.float32),
        mesh=mesh,
        compiler_params=pltpu.CompilerParams(needs_layout_passes=False),
        scratch_types=[
            pltpu.VMEM((_L * _D,), jnp.float32),
            pltpu.VMEM((2 * _RPW,), jnp.float32),
            pltpu.VMEM((_RPW + _LANES,), jnp.float32),
            pltpu.VMEM((_RPW + _LANES,), jnp.float32),
            pltpu.VMEM((_RPW + _LANES,), jnp.int32),
            pltpu.VMEM((2 * _D,), jnp.float32),
            pltpu.VMEM((_D,), jnp.float32),
            pltpu.VMEM((_BLK, _D), jnp.float32),
            pltpu.VMEM((_BLK, _D), jnp.float32),
            pltpu.SemaphoreType.DMA,
            pltpu.SemaphoreType.DMA,
        ],
    )
    out = enc(xy, lab, tab, w, b)
    return out.reshape(_B, _P, _D)


# speedup vs baseline: 1.5728x; 1.5728x over previous
import jax
import jax.numpy as jnp
from jax import lax
from jax.experimental import pallas as pl
from jax.experimental.pallas import tpu as pltpu
from jax.experimental.pallas import tpu_sc as plsc

_B, _P, _D, _L = 64, 1024, 256, 80
_N = _B * _P
_NC, _NS = 2, 16
_NW = _NC * _NS
_RPW = _N // _NW
_BLK = 128
_NBLK = _RPW // _BLK
_LANES = 16


def _sc_encoder(x_hbm, y_hbm, lab_hbm, tab_hbm, w0_hbm, w1_hbm, out_hbm,
                tab_v, x_v, y_v, lab_v, w0_v, w1_v, out_v0, out_v1,
                sem0, sem1):
    wid = lax.axis_index("s") * _NC + lax.axis_index("c")
    base = wid * _RPW

    pltpu.sync_copy(x_hbm.at[pl.ds(base, _RPW)], x_v.at[pl.ds(0, _RPW)])
    pltpu.sync_copy(y_hbm.at[pl.ds(base, _RPW)], y_v.at[pl.ds(0, _RPW)])
    pltpu.sync_copy(lab_hbm.at[pl.ds(base, _RPW)], lab_v.at[pl.ds(0, _RPW)])
    pltpu.sync_copy(tab_hbm, tab_v)
    pltpu.sync_copy(w0_hbm, w0_v)
    pltpu.sync_copy(w1_hbm, w1_v)

    w0r = [w0_v[pl.ds(jb * _LANES, _LANES)] for jb in range(_D // _LANES)]
    w1r = [w1_v[pl.ds(jb * _LANES, _LANES)] for jb in range(_D // _LANES)]

    def compute_block(buf, off):
        @plsc.parallel_loop(0, _BLK, unroll=2)
        def body(i):
            p = i + off
            xv = x_v[pl.ds(p, _LANES)]
            yv = y_v[pl.ds(p, _LANES)]
            lv = lab_v[pl.ds(p, _LANES)]
            xs = xv[0]
            ys = yv[0]
            rowbase = lv[0]
            for jb in range(_D // _LANES):
                tbl = tab_v[pl.ds(rowbase + jb * _LANES, _LANES)]
                val = tbl + xs * w0r[jb] + ys * w1r[jb]
                buf[i, pl.ds(jb * _LANES, _LANES)] = val

    def issue(buf, off, sem):
        pltpu.async_copy(buf, out_hbm.at[pl.ds(base + off, _BLK)], sem)

    def drain(buf, off, sem):
        pltpu.make_async_copy(buf, out_hbm.at[pl.ds(base + off, _BLK)], sem).wait()

    compute_block(out_v0, 0)
    issue(out_v0, 0, sem0)
    compute_block(out_v1, _BLK)
    issue(out_v1, _BLK, sem1)

    def sb_body(sb, carry):
        off = sb * (2 * _BLK)
        drain(out_v0, off, sem0)
        compute_block(out_v0, off)
        issue(out_v0, off, sem0)
        drain(out_v1, off + _BLK, sem1)
        compute_block(out_v1, off + _BLK)
        issue(out_v1, off + _BLK, sem1)
        return carry

    lax.fori_loop(1, _NBLK // 2, sb_body, 0)
    drain(out_v0, 0, sem0)
    drain(out_v1, _BLK, sem1)


def kernel(points, labels, W_pos, b_pos, label_table):
    x = points[:, :, 0].reshape(-1).astype(jnp.float32)
    y = points[:, :, 1].reshape(-1).astype(jnp.float32)
    lab256 = labels.reshape(-1).astype(jnp.int32) * _D
    tab2 = (label_table + b_pos[None, :]).reshape(-1).astype(jnp.float32)
    w0 = W_pos[0].astype(jnp.float32)
    w1 = W_pos[1].astype(jnp.float32)

    mesh = plsc.VectorSubcoreMesh(core_axis_name="c", subcore_axis_name="s")
    enc = pl.kernel(
        _sc_encoder,
        out_type=jax.ShapeDtypeStruct((_N, _D), jnp.float32),
        mesh=mesh,
        compiler_params=pltpu.CompilerParams(needs_layout_passes=False),
        scratch_types=[
            pltpu.VMEM((_L * _D,), jnp.float32),
            pltpu.VMEM((_RPW + _LANES,), jnp.float32),
            pltpu.VMEM((_RPW + _LANES,), jnp.float32),
            pltpu.VMEM((_RPW + _LANES,), jnp.int32),
            pltpu.VMEM((_D,), jnp.float32),
            pltpu.VMEM((_D,), jnp.float32),
            pltpu.VMEM((_BLK, _D), jnp.float32),
            pltpu.VMEM((_BLK, _D), jnp.float32),
            pltpu.SemaphoreType.DMA,
            pltpu.SemaphoreType.DMA,
        ],
    )
    out = enc(x, y, lab256, tab2, w0, w1)
    return out.reshape(_B, _P, _D)
